# baseline (speedup 1.0000x reference)
import functools

import jax
import jax.numpy as jnp
from jax import lax
from jax.experimental import pallas as pl
from jax.experimental.pallas import tpu as pltpu
from jax.experimental.pallas import tpu_sc as plsc


def _gather_kernel(NB, S, D):
    info = plsc.get_sparse_core_info()
    NC, NS, L = info.num_cores, info.num_subcores, info.num_lanes
    NW = NC * NS
    IPW = NB // NW
    HB = IPW // 2
    G = D // 8
    NIB = NB // 128
    n_steps = 2 * S
    mesh = plsc.VectorSubcoreMesh(core_axis_name="c", subcore_axis_name="s")

    @functools.partial(
        pl.kernel,
        mesh=mesh,
        out_type=jax.ShapeDtypeStruct((S, G, NIB, 8, 128), jnp.float32),
        scratch_types=[
            pltpu.VMEM((IPW, S), jnp.int32),
            pltpu.VMEM((HB,), jnp.int32),
            pltpu.VMEM((HB,), jnp.int32),
            pltpu.VMEM((HB, D), jnp.float32),
            pltpu.VMEM((HB, D), jnp.float32),
            pltpu.VMEM((G, 2, 8, 128), jnp.float32),
            pltpu.VMEM((G, 2, 8, 128), jnp.float32),
            pltpu.SemaphoreType.DMA((2,)),
            pltpu.SemaphoreType.DMA((2,)),
        ],
        compiler_params=pltpu.CompilerParams(
            use_tc_tiling_on_sc=False, needs_layout_passes=False),
    )
    def k(elem_hbm, table_hbm, out_hbm, ebuf, ib0, ib1, ra, rb, ta, tb,
          sem_g, sem_o):
        wid = lax.axis_index("s") * NC + lax.axis_index("c")
        i0 = wid * IPW
        ibufs, rbufs, tbufs = (ib0, ib1), (ra, rb), (ta, tb)
        iota = lax.broadcasted_iota(jnp.int32, (L,), 0)

        pltpu.sync_copy(elem_hbm.at[pl.ds(i0, IPW)], ebuf)

        def build_idx(j, half, sl):
            jvec = jnp.full((L,), 0, jnp.int32) + j

            def body(ig, carry):
                ivec = ig * L + iota + half * HB
                v = plsc.load_gather(ebuf, [ivec, jvec])
                ibufs[sl][pl.ds(ig * L, L)] = v
                return carry

            lax.fori_loop(0, HB // L, body, 0)

        def gat(sl):
            return pltpu.make_async_copy(
                table_hbm.at[ibufs[sl]], rbufs[sl], sem_g.at[sl])

        def transpose(sl):
            def body(ig, carry):
                ivec = ig * L + iota
                iblk = ig // 8
                il0 = (ig % 8) * L
                for d in range(D):
                    dvec = jnp.full((L,), d, jnp.int32)
                    v = plsc.load_gather(rbufs[sl], [ivec, dvec])
                    tbufs[sl][d // 8, iblk, d % 8, pl.ds(il0, L)] = v
                return carry

            lax.fori_loop(0, HB // L, body, 0)

        def sto(j, half, sl, g):
            ib = wid * 4 + half * 2
            return pltpu.make_async_copy(
                tbufs[sl].at[g],
                out_hbm.at[j, g, pl.ds(ib, 2)],
                sem_o.at[sl])

        def sto_start(j, half, sl):
            for g in range(G):
                sto(j, half, sl, g).start()

        def sto_wait(j, half, sl):
            for g in range(G):
                sto(j, half, sl, g).wait()

        build_idx(0, 0, 0)
        gat(0).start()
        build_idx(0, 1, 1)
        gat(1).start()

        for h in range(2):
            gat(h).wait()
            transpose(h)
            build_idx(1, h, h)
            gat(h).start()
            sto_start(0, h, h)

        def outer(j, carry):
            for h in range(2):
                gat(h).wait()
                sto_wait(j - 1, h, h)
                transpose(h)
                build_idx(j + 1, h, h)
                gat(h).start()
                sto_start(j, h, h)
            return carry

        lax.fori_loop(1, S - 1, outer, 0)

        for h in range(2):
            gat(h).wait()
            sto_wait(S - 2, h, h)
            transpose(h)
            sto_start(S - 1, h, h)
        for h in range(2):
            sto_wait(S - 1, h, h)

    return k


def kernel(element, table):
    NB, S = element.shape
    V, D = table.shape
    t5 = _gather_kernel(NB, S, D)(element, table)
    return t5.transpose(2, 4, 0, 1, 3).reshape(NB, S, D)


# speedup vs baseline: 1.9816x; 1.9816x over previous
import functools

import jax
import jax.numpy as jnp
from jax import lax
from jax.experimental import pallas as pl
from jax.experimental.pallas import tpu as pltpu
from jax.experimental.pallas import tpu_sc as plsc

_NBUF = 4


def _gather_kernel(NB, S, D, BPC):
    SP = (S + 7) // 8 * 8
    DP = (D + 127) // 128 * 128
    CH = BPC * S
    info = plsc.get_sparse_core_info()
    NC, NS = info.num_cores, info.num_subcores
    NW = NC * NS
    nb_per_w = NB // NW
    n_ch = nb_per_w // BPC
    n_outer = n_ch // _NBUF
    assert nb_per_w % BPC == 0 and n_ch % _NBUF == 0 and n_outer >= 3
    mesh = plsc.VectorSubcoreMesh(core_axis_name="c", subcore_axis_name="s")

    @functools.partial(
        pl.kernel,
        mesh=mesh,
        out_type=jax.ShapeDtypeStruct((NB, SP, DP), jnp.float32),
        scratch_types=[
            pltpu.VMEM((n_ch, CH), jnp.int32),
            pltpu.VMEM((_NBUF, CH, D), jnp.float32),
            pltpu.SemaphoreType.DMA((_NBUF,)),
            pltpu.SemaphoreType.DMA((_NBUF,)),
        ],
        compiler_params=pltpu.CompilerParams(use_tc_tiling_on_sc=False),
    )
    def k(idx_hbm, table_hbm, out_hbm, idx_v, rows_v, sem_g, sem_o):
        wid = lax.axis_index("s") * NC + lax.axis_index("c")
        c0 = wid * n_ch

        pltpu.sync_copy(idx_hbm.at[pl.ds(c0, n_ch)], idx_v)

        def gat(i, b):
            return pltpu.make_async_copy(
                table_hbm.at[idx_v.at[i]], rows_v.at[b], sem_g.at[b])

        def _sto(i, b, kq):
            return pltpu.make_async_copy(
                rows_v.at[b, pl.ds(kq * S, S)],
                out_hbm.at[(c0 + i) * BPC + kq, pl.ds(0, S), pl.ds(0, D)],
                sem_o.at[b])

        def sto_start(i, b):
            for kq in range(BPC):
                _sto(i, b, kq).start()

        def sto_wait(i, b):
            for kq in range(BPC):
                _sto(i, b, kq).wait()

        for b in range(_NBUF - 1):
            gat(b, b).start()

        for b in range(_NBUF):
            gat(b, b).wait()
            sto_start(b, b)
            if b > 0:
                sto_wait(b - 1, b - 1)
            gat(b + _NBUF - 1, (b + _NBUF - 1) % _NBUF).start()

        def outer(g, carry):
            i0 = g * _NBUF
            for b in range(_NBUF):
                i = i0 + b
                gat(i, b).wait()
                sto_start(i, b)
                bp = (b - 1) % _NBUF
                sto_wait(i - 1, bp)
                gat(i + _NBUF - 1, bp).start()
            return carry

        lax.fori_loop(1, n_outer - 1, outer, 0)

        i0 = (n_outer - 1) * _NBUF
        for b in range(_NBUF):
            i = i0 + b
            gat(i, b).wait()
            sto_start(i, b)
            bp = (b - 1) % _NBUF
            sto_wait(i - 1, bp)
            if b == 0:
                gat(i + _NBUF - 1, bp).start()
        sto_wait(i0 + _NBUF - 1, _NBUF - 1)

    return k


def kernel(element, table):
    NB, S = element.shape
    V, D = table.shape
    BPC = 4
    idx = element.reshape(NB // BPC, BPC * S)
    zero = (element[0, 0] & 0).astype(table.dtype)
    padded = _gather_kernel(NB, S, D, BPC)(idx, table + zero)
    return padded[:, :S, :D]
